# Initial kernel scaffold; baseline (speedup 1.0000x reference)
import jax
import jax.numpy as jnp
from jax.experimental import pallas as pl


def kernel(x, edge_index, W1, W2):
    raise NotImplementedError("write your pallas kernel here")


# speedup vs baseline: 25.5342x; 25.5342x over previous
import functools

import jax
import jax.numpy as jnp
from jax import lax
from jax.experimental import pallas as pl
from jax.experimental.pallas import tpu as pltpu
from jax.experimental.pallas import tpu_sc as plsc

N_NODES = 10000
N_PAD = 10240
N_EDGES = 320000
NC, NS = 2, 16
NW = NC * NS
CH = 128
CPW = 79
E_PAD = NW * CPW * CH
D_HID = 32
STRIPE = N_PAD // NS


def _sc_mesh():
    return plsc.VectorSubcoreMesh(core_axis_name="c", subcore_axis_name="s")


_SC_PARAMS = pltpu.CompilerParams(use_tc_tiling_on_sc=False)


def _deg_body(dst_hbm, zer_hbm, out_hbm, idx_v, ones_v, deg_sh, sem):
    c = lax.axis_index("c")
    s = lax.axis_index("s")
    wid = c * NS + s
    pltpu.sync_copy(zer_hbm.at[pl.ds(s * STRIPE, STRIPE)],
                    deg_sh.at[pl.ds(s * STRIPE, STRIPE)])
    pltpu.async_copy(dst_hbm.at[wid], idx_v, sem).wait()
    for i in range(CH // 16):
        ones_v[pl.ds(i * 16, 16)] = jnp.full((16,), 1.0, jnp.float32)
    plsc.subcore_barrier()

    def body(j, carry):
        pltpu.sync_copy(ones_v, deg_sh.at[idx_v.at[j]], add=True)
        return carry

    lax.fori_loop(0, CPW, body, 0)
    plsc.subcore_barrier()
    pltpu.sync_copy(deg_sh.at[pl.ds(s * STRIPE, STRIPE)],
                    out_hbm.at[c, pl.ds(s * STRIPE, STRIPE)])


def _make_deg_kernel():
    return pl.kernel(
        _deg_body,
        out_type=jax.ShapeDtypeStruct((NC, N_PAD), jnp.float32),
        mesh=_sc_mesh(),
        scratch_types=[
            pltpu.VMEM((CPW, CH), jnp.int32),
            pltpu.VMEM((CH,), jnp.float32),
            pltpu.VMEM_SHARED((N_PAD,), jnp.float32),
            pltpu.SemaphoreType.DMA,
        ],
        compiler_params=_SC_PARAMS,
    )


def _prop_body(tab_hbm, src_hbm, dst_hbm, zer_hbm, out_hbm,
               sidx_v, didx_v, rows_v, acc_sh, sem):
    c = lax.axis_index("c")
    s = lax.axis_index("s")
    wid = c * NS + s
    pltpu.sync_copy(zer_hbm.at[pl.ds(s * STRIPE, STRIPE)],
                    acc_sh.at[pl.ds(s * STRIPE, STRIPE)])
    pltpu.async_copy(src_hbm.at[wid], sidx_v, sem).wait()
    pltpu.async_copy(dst_hbm.at[wid], didx_v, sem).wait()
    plsc.subcore_barrier()

    def body(j, carry):
        pltpu.async_copy(tab_hbm.at[sidx_v.at[j]], rows_v, sem).wait()
        pltpu.sync_copy(rows_v, acc_sh.at[didx_v.at[j]], add=True)
        return carry

    lax.fori_loop(0, CPW, body, 0)
    plsc.subcore_barrier()
    pltpu.sync_copy(acc_sh.at[pl.ds(s * STRIPE, STRIPE)],
                    out_hbm.at[c, pl.ds(s * STRIPE, STRIPE)])


def _make_prop_kernel():
    return pl.kernel(
        _prop_body,
        out_type=jax.ShapeDtypeStruct((NC, N_PAD, D_HID), jnp.float32),
        mesh=_sc_mesh(),
        scratch_types=[
            pltpu.VMEM((CPW, CH), jnp.int32),
            pltpu.VMEM((CPW, CH), jnp.int32),
            pltpu.VMEM((CH, D_HID), jnp.float32),
            pltpu.VMEM_SHARED((N_PAD, D_HID), jnp.float32),
            pltpu.SemaphoreType.DMA,
        ],
        compiler_params=_SC_PARAMS,
    )


BR = 1024


def _dinv_of(dp_ref):
    deg = dp_ref[0, :] + dp_ref[1, :] + 1.0
    return lax.rsqrt(deg)


def _h1s_body(x_ref, w_ref, dp_ref, o_ref):
    dinv = _dinv_of(dp_ref)
    h = jnp.dot(x_ref[...], w_ref[...], preferred_element_type=jnp.float32)
    o_ref[...] = h * dinv[:, None]


def _mid_body(s1_ref, h1s_ref, dp_ref, o_ref):
    dinv = _dinv_of(dp_ref)
    t = (s1_ref[0] + s1_ref[1] + h1s_ref[...]) * dinv[:, None]
    o_ref[...] = jnp.tanh(t) * dinv[:, None]


def _out_body(s2_ref, h2s_ref, dp_ref, w2_ref, o_ref):
    dinv = _dinv_of(dp_ref)
    p = (s2_ref[0] + s2_ref[1] + h2s_ref[...]) * dinv[:, None]
    o_ref[...] = jnp.dot(p, w2_ref[...], preferred_element_type=jnp.float32)


def _row_spec(d):
    return pl.BlockSpec((BR, d), lambda i: (i, 0))


def _part_spec(d):
    return pl.BlockSpec((NC, BR, d), lambda i: (0, i, 0))


_DP_SPEC = pl.BlockSpec((NC, BR), lambda i: (0, i))
_GRID = (N_PAD // BR,)


def _h1s_call(xp, W1, deg_part):
    return pl.pallas_call(
        _h1s_body,
        grid=_GRID,
        in_specs=[_row_spec(128),
                  pl.BlockSpec((128, D_HID), lambda i: (0, 0)),
                  _DP_SPEC],
        out_specs=_row_spec(D_HID),
        out_shape=jax.ShapeDtypeStruct((N_PAD, D_HID), jnp.float32),
    )(xp, W1, deg_part)


def _mid_call(s1, h1s, deg_part):
    return pl.pallas_call(
        _mid_body,
        grid=_GRID,
        in_specs=[_part_spec(D_HID), _row_spec(D_HID), _DP_SPEC],
        out_specs=_row_spec(D_HID),
        out_shape=jax.ShapeDtypeStruct((N_PAD, D_HID), jnp.float32),
    )(s1, h1s, deg_part)


def _out_call(s2, h2s, deg_part, W2p):
    return pl.pallas_call(
        _out_body,
        grid=_GRID,
        in_specs=[_part_spec(D_HID), _row_spec(D_HID), _DP_SPEC,
                  pl.BlockSpec((D_HID, 128), lambda i: (0, 0))],
        out_specs=_row_spec(128),
        out_shape=jax.ShapeDtypeStruct((N_PAD, 128), jnp.float32),
    )(s2, h2s, deg_part, W2p)


def kernel(x, edge_index, W1, W2):
    n = x.shape[0]
    pad_e = E_PAD - N_EDGES
    dummy = jnp.full((pad_e,), n, dtype=jnp.int32)
    srcp = jnp.concatenate([edge_index[0], dummy]).reshape(NW, CPW, CH)
    dstp = jnp.concatenate([edge_index[1], dummy]).reshape(NW, CPW, CH)
    xp = jnp.pad(x, ((0, N_PAD - n), (0, 0)))
    W2p = jnp.pad(W2, ((0, 0), (0, 128 - W2.shape[1])))
    zer1 = jnp.zeros((N_PAD,), jnp.float32)
    zer2 = jnp.zeros((N_PAD, D_HID), jnp.float32)

    deg_part = _make_deg_kernel()(dstp, zer1)
    h1s = _h1s_call(xp, W1, deg_part)
    prop = _make_prop_kernel()
    s1 = prop(h1s, srcp, dstp, zer2)
    h2s = _mid_call(s1, h1s, deg_part)
    s2 = prop(h2s, srcp, dstp, zer2)
    zp = _out_call(s2, h2s, deg_part, W2p)
    z = zp[:n, :W2.shape[1]]
    return (z, z)


# speedup vs baseline: 28.8878x; 1.1313x over previous
import functools

import jax
import jax.numpy as jnp
from jax import lax
from jax.experimental import pallas as pl
from jax.experimental.pallas import tpu as pltpu
from jax.experimental.pallas import tpu_sc as plsc

N_NODES = 10000
N_PAD = 10240
N_EDGES = 320000
NC, NS = 2, 16
NW = NC * NS
CH = 128
CPW = 80
E_PAD = NW * CPW * CH
D_HID = 32
STRIPE = N_PAD // NS


def _sc_mesh():
    return plsc.VectorSubcoreMesh(core_axis_name="c", subcore_axis_name="s")


_SC_PARAMS = pltpu.CompilerParams(use_tc_tiling_on_sc=False)


def _deg_body(dst_hbm, zer_hbm, out_hbm, idx_v, ones_v, deg_sh, sem):
    c = lax.axis_index("c")
    s = lax.axis_index("s")
    wid = c * NS + s
    pltpu.sync_copy(zer_hbm.at[pl.ds(s * STRIPE, STRIPE)],
                    deg_sh.at[pl.ds(s * STRIPE, STRIPE)])
    pltpu.async_copy(dst_hbm.at[wid], idx_v, sem).wait()
    for i in range(CH // 16):
        ones_v[pl.ds(i * 16, 16)] = jnp.full((16,), 1.0, jnp.float32)
    plsc.subcore_barrier()

    def body(j, carry):
        pltpu.sync_copy(ones_v, deg_sh.at[idx_v.at[j]], add=True)
        return carry

    lax.fori_loop(0, CPW, body, 0)
    plsc.subcore_barrier()
    pltpu.sync_copy(deg_sh.at[pl.ds(s * STRIPE, STRIPE)],
                    out_hbm.at[c, pl.ds(s * STRIPE, STRIPE)])


def _make_deg_kernel():
    return pl.kernel(
        _deg_body,
        out_type=jax.ShapeDtypeStruct((NC, N_PAD), jnp.float32),
        mesh=_sc_mesh(),
        scratch_types=[
            pltpu.VMEM((CPW, CH), jnp.int32),
            pltpu.VMEM((CH,), jnp.float32),
            pltpu.VMEM_SHARED((N_PAD,), jnp.float32),
            pltpu.SemaphoreType.DMA,
        ],
        compiler_params=_SC_PARAMS,
    )


def _prop_body(tab_hbm, src_hbm, dst_hbm, zer_hbm, out_hbm,
               sidx_v, didx_v, rows0_v, rows1_v, acc_sh, sem0, sem1):
    c = lax.axis_index("c")
    s = lax.axis_index("s")
    wid = c * NS + s
    pltpu.sync_copy(zer_hbm.at[pl.ds(s * STRIPE, STRIPE)],
                    acc_sh.at[pl.ds(s * STRIPE, STRIPE)])
    pltpu.async_copy(src_hbm.at[wid], sidx_v, sem0).wait()
    pltpu.async_copy(dst_hbm.at[wid], didx_v, sem0).wait()
    plsc.subcore_barrier()

    pltpu.async_copy(tab_hbm.at[sidx_v.at[0]], rows0_v, sem0)
    pltpu.async_copy(tab_hbm.at[sidx_v.at[1]], rows1_v, sem1)

    def body(k, carry):
        a = 2 * k
        pltpu.make_async_copy(tab_hbm.at[sidx_v.at[a]], rows0_v, sem0).wait()
        pltpu.sync_copy(rows0_v, acc_sh.at[didx_v.at[a]], add=True)

        @pl.when(k < CPW // 2 - 1)
        def _():
            pltpu.async_copy(tab_hbm.at[sidx_v.at[a + 2]], rows0_v, sem0)

        pltpu.make_async_copy(tab_hbm.at[sidx_v.at[a + 1]], rows1_v, sem1).wait()
        pltpu.sync_copy(rows1_v, acc_sh.at[didx_v.at[a + 1]], add=True)

        @pl.when(k < CPW // 2 - 1)
        def _():
            pltpu.async_copy(tab_hbm.at[sidx_v.at[a + 3]], rows1_v, sem1)

        return carry

    lax.fori_loop(0, CPW // 2, body, 0)
    plsc.subcore_barrier()
    pltpu.sync_copy(acc_sh.at[pl.ds(s * STRIPE, STRIPE)],
                    out_hbm.at[c, pl.ds(s * STRIPE, STRIPE)])


def _make_prop_kernel():
    return pl.kernel(
        _prop_body,
        out_type=jax.ShapeDtypeStruct((NC, N_PAD, D_HID), jnp.float32),
        mesh=_sc_mesh(),
        scratch_types=[
            pltpu.VMEM((CPW, CH), jnp.int32),
            pltpu.VMEM((CPW, CH), jnp.int32),
            pltpu.VMEM((CH, D_HID), jnp.float32),
            pltpu.VMEM((CH, D_HID), jnp.float32),
            pltpu.VMEM_SHARED((N_PAD, D_HID), jnp.float32),
            pltpu.SemaphoreType.DMA,
            pltpu.SemaphoreType.DMA,
        ],
        compiler_params=_SC_PARAMS,
    )


BR = 1024


def _dinv_of(dp_ref):
    deg = dp_ref[0, :] + dp_ref[1, :] + 1.0
    return lax.rsqrt(deg)


def _h1s_body(x_ref, w_ref, dp_ref, o_ref):
    dinv = _dinv_of(dp_ref)
    h = jnp.dot(x_ref[...], w_ref[...], preferred_element_type=jnp.float32)
    o_ref[...] = h * dinv[:, None]


def _mid_body(s1_ref, h1s_ref, dp_ref, o_ref):
    dinv = _dinv_of(dp_ref)
    t = (s1_ref[0] + s1_ref[1] + h1s_ref[...]) * dinv[:, None]
    o_ref[...] = jnp.tanh(t) * dinv[:, None]


def _out_body(s2_ref, h2s_ref, dp_ref, w2_ref, o_ref):
    dinv = _dinv_of(dp_ref)
    p = (s2_ref[0] + s2_ref[1] + h2s_ref[...]) * dinv[:, None]
    o_ref[...] = jnp.dot(p, w2_ref[...], preferred_element_type=jnp.float32)


def _row_spec(d):
    return pl.BlockSpec((BR, d), lambda i: (i, 0))


def _part_spec(d):
    return pl.BlockSpec((NC, BR, d), lambda i: (0, i, 0))


_DP_SPEC = pl.BlockSpec((NC, BR), lambda i: (0, i))
_GRID = (N_PAD // BR,)


def _h1s_call(xp, W1, deg_part):
    return pl.pallas_call(
        _h1s_body,
        grid=_GRID,
        in_specs=[_row_spec(128),
                  pl.BlockSpec((128, D_HID), lambda i: (0, 0)),
                  _DP_SPEC],
        out_specs=_row_spec(D_HID),
        out_shape=jax.ShapeDtypeStruct((N_PAD, D_HID), jnp.float32),
    )(xp, W1, deg_part)


def _mid_call(s1, h1s, deg_part):
    return pl.pallas_call(
        _mid_body,
        grid=_GRID,
        in_specs=[_part_spec(D_HID), _row_spec(D_HID), _DP_SPEC],
        out_specs=_row_spec(D_HID),
        out_shape=jax.ShapeDtypeStruct((N_PAD, D_HID), jnp.float32),
    )(s1, h1s, deg_part)


def _out_call(s2, h2s, deg_part, W2p):
    return pl.pallas_call(
        _out_body,
        grid=_GRID,
        in_specs=[_part_spec(D_HID), _row_spec(D_HID), _DP_SPEC,
                  pl.BlockSpec((D_HID, 128), lambda i: (0, 0))],
        out_specs=_row_spec(128),
        out_shape=jax.ShapeDtypeStruct((N_PAD, 128), jnp.float32),
    )(s2, h2s, deg_part, W2p)


def kernel(x, edge_index, W1, W2):
    n = x.shape[0]
    pad_e = E_PAD - N_EDGES
    dummy = jnp.full((pad_e,), n, dtype=jnp.int32)
    srcp = jnp.concatenate([edge_index[0], dummy]).reshape(NW, CPW, CH)
    dstp = jnp.concatenate([edge_index[1], dummy]).reshape(NW, CPW, CH)
    xp = jnp.pad(x, ((0, N_PAD - n), (0, 0)))
    W2p = jnp.pad(W2, ((0, 0), (0, 128 - W2.shape[1])))
    zer1 = jnp.zeros((N_PAD,), jnp.float32)
    zer2 = jnp.zeros((N_PAD, D_HID), jnp.float32)

    deg_part = _make_deg_kernel()(dstp, zer1)
    h1s = _h1s_call(xp, W1, deg_part)
    prop = _make_prop_kernel()
    s1 = prop(h1s, srcp, dstp, zer2)
    h2s = _mid_call(s1, h1s, deg_part)
    s2 = prop(h2s, srcp, dstp, zer2)
    zp = _out_call(s2, h2s, deg_part, W2p)
    z = zp[:n, :W2.shape[1]]
    return (z, z)
